# baseline (speedup 1.0000x reference)
import jax
import jax.numpy as jnp
from jax import lax
from jax.experimental import pallas as pl
from jax.experimental.pallas import tpu as pltpu

_N = 128
_B = 8
_HID = 128
_HEADS = 8
_HD = 16
_L = 4
_BN = _B * _N
_JT = 64


def _ln(h, g, b, ones_col):
    f32 = jnp.float32
    mu = lax.dot(h, ones_col, preferred_element_type=f32) * (1.0 / _HID)
    ex2 = lax.dot(h * h, ones_col, preferred_element_type=f32) * (1.0 / _HID)
    v = ex2 - mu * mu
    return (h - mu) * lax.rsqrt(v + 1e-5) * g + b


def _fwd(x_ref, ee_ref, w1_ref, b1_ref, g1_ref, bb1_ref, w2_ref, b2_ref,
         g2_ref, bb2_ref, llw_ref, llb_ref, lrw_ref, lrb_ref, lew_ref,
         att_ref, cb_ref, pw_ref, pb_ref, lg_ref, lb_ref, ow_ref, ob_ref,
         out_ref, agg_ref, xl_ref, xr_ref):
    f32 = jnp.float32

    def dot(a, b):
        return lax.dot(a, b, preferred_element_type=f32)

    ones_col = jnp.ones((_HID, 1), f32)
    h = _ln(dot(x_ref[:], w1_ref[:]) + b1_ref[:], g1_ref[:], bb1_ref[:],
            ones_col)
    h = jnp.maximum(h, 0.0)
    h = _ln(dot(h, w2_ref[:]) + b2_ref[:], g2_ref[:], bb2_ref[:], ones_col)

    hm_h = lax.broadcasted_iota(jnp.int32, (_HEADS, _HID), 0)
    hm_c = lax.broadcasted_iota(jnp.int32, (_HEADS, _HID), 1) // _HD
    hexpand = (hm_h == hm_c).astype(f32)

    def dott(a, b):
        return lax.dot_general(a, b, (((1,), (1,)), ((), ())),
                               preferred_element_type=f32)

    for l in range(_L):
        h_prev = h
        xl_ref[...] = dot(h, llw_ref[l]) + llb_ref[l:l + 1, :]
        xr_ref[...] = dot(h, lrw_ref[l]) + lrb_ref[l:l + 1, :]
        etab = dot(ee_ref[:], lew_ref[l])
        catt = hexpand * att_ref[l:l + 1, :]
        catt_bf = catt.astype(jnp.bfloat16)

        def b_body(b, _, etab=etab, catt_bf=catt_bf):
            xl_b = xl_ref[pl.ds(b * _N, _N), :]
            irow = lax.broadcasted_iota(jnp.int32, (_N, 1), 0)
            lo_i = irow < (_N // 2)
            xlh = jnp.concatenate(
                [xl_b * hexpand[hh:hh + 1, :] for hh in range(_HEADS)],
                axis=0)

            for t in range(_N // _JT):
                j0 = t * _JT
                bf = jnp.bfloat16
                xr_t = xr_ref[pl.ds(b * _N + j0, _JT), :]
                xr_bf = xr_t.astype(bf)
                xlE = jnp.where(lo_i, xl_b + etab[t:t + 1, :],
                                xl_b + etab[2 + t:3 + t, :])
                xlE_bf = xlE.astype(bf)
                s = xr_bf[:, None, :] + xlE_bf[None, :, :]
                m = jnp.maximum(s, bf(0.2) * s)
                lt = dott(catt_bf, m.reshape(_JT * _N, _HID))
                lt3 = lt.reshape(_HEADS, _JT, _N)
                xl_t = xl_ref[pl.ds(b * _N + j0, _JT), :]
                sdr = xr_bf + (xl_t + etab[4 + t:5 + t, :]).astype(bf)
                mdr = jnp.maximum(sdr, bf(0.2) * sdr)
                sdw = xr_bf + (xl_t + etab[3 * t:3 * t + 1, :]).astype(bf)
                mdw = jnp.maximum(sdw, bf(0.2) * sdw)
                ldr = dott(catt_bf, mdr)
                ldw = dott(catt_bf, mdw)
                amax = jnp.maximum(jnp.max(lt3, axis=2), ldr)
                ex = jnp.exp(lt3 - amax[:, :, None])
                edr = jnp.exp(ldr - amax)
                edw = jnp.exp(ldw - amax)
                den = jnp.sum(ex, axis=2) - edw + edr
                rec = 1.0 / (den + 1e-16)
                alpha = ex * rec[:, :, None]
                alpha_cat = jnp.concatenate(
                    [alpha[hh] for hh in range(_HEADS)], axis=1)
                agg_t = dot(alpha_cat, xlh)
                dal = (edr - edw) * rec
                agg_t = agg_t + dot(dal.T, hexpand) * xl_t
                agg_ref[pl.ds(b * _N + j0, _JT), :] = agg_t
            return 0

        lax.fori_loop(0, _B, b_body, 0, unroll=_B)
        hh = agg_ref[:] + cb_ref[l:l + 1, :]
        hh = dot(hh, pw_ref[l]) + pb_ref[l:l + 1, :]
        hh = _ln(hh, lg_ref[l:l + 1, :], lb_ref[l:l + 1, :], ones_col)
        hh = jnp.maximum(hh, 0.0)
        h = hh + h_prev

    hag = jnp.sum(h.reshape(_B, _N, _HID), axis=1)
    out_ref[:] = dot(hag, ow_ref[:]) + ob_ref[:]


def kernel(x, edge_emb, mlp_w1, mlp_b1, ln1_g, ln1_b, mlp_w2, mlp_b2, ln2_g,
           ln2_b, lin_l_w, lin_l_b, lin_r_w, lin_r_b, lin_e_w, att, conv_bias,
           proj_w, proj_b, lnl_g, lnl_b, out_w, out_b):
    f32 = jnp.float32
    x2 = x.reshape(_BN, 2).astype(f32)
    att_flat = att.reshape(_L, _HID)

    def row(v):
        return v.reshape(1, -1)

    return pl.pallas_call(
        _fwd,
        out_shape=jax.ShapeDtypeStruct((_B, 1), f32),
        scratch_shapes=[pltpu.VMEM((_BN, _HID), f32),
                        pltpu.VMEM((_BN, _HID), f32),
                        pltpu.VMEM((_BN, _HID), f32)],
    )(x2, edge_emb, mlp_w1, row(mlp_b1), row(ln1_g), row(ln1_b), mlp_w2,
      row(mlp_b2), row(ln2_g), row(ln2_b), lin_l_w, lin_l_b, lin_r_w,
      lin_r_b, lin_e_w, att_flat, conv_bias, proj_w, proj_b, lnl_g, lnl_b,
      out_w, out_b.reshape(1, 1))


# speedup vs baseline: 1.1992x; 1.1992x over previous
import jax
import jax.numpy as jnp
from jax import lax
from jax.experimental import pallas as pl
from jax.experimental.pallas import tpu as pltpu

_N = 128
_B = 8
_HID = 128
_HEADS = 8
_HD = 16
_L = 4
_BN = _B * _N
_JT = 64


def _ln(h, g, b):
    mu = jnp.mean(h, axis=-1, keepdims=True)
    d = h - mu
    v = jnp.mean(d * d, axis=-1, keepdims=True)
    return d * lax.rsqrt(v + 1e-5) * g + b


def _fwd(x_ref, ee_ref, w1_ref, b1_ref, g1_ref, bb1_ref, w2_ref, b2_ref,
         g2_ref, bb2_ref, llw_ref, llb_ref, lrw_ref, lrb_ref, lew_ref,
         att_ref, cb_ref, pw_ref, pb_ref, lg_ref, lb_ref, ow_ref, ob_ref,
         out_ref, agg_ref, xl_ref, xr_ref):
    f32 = jnp.float32

    def dot(a, b):
        return lax.dot(a, b, preferred_element_type=f32)

    h = _ln(dot(x_ref[:], w1_ref[:]) + b1_ref[:], g1_ref[:], bb1_ref[:])
    h = jnp.maximum(h, 0.0)
    h = _ln(dot(h, w2_ref[:]) + b2_ref[:], g2_ref[:], bb2_ref[:])

    hm_h = lax.broadcasted_iota(jnp.int32, (_HEADS, _HID), 0)
    hm_c = lax.broadcasted_iota(jnp.int32, (_HEADS, _HID), 1) // _HD
    hexpand = (hm_h == hm_c).astype(f32)

    def dott(a, b):
        return lax.dot_general(a, b, (((1,), (1,)), ((), ())),
                               preferred_element_type=f32)

    for l in range(_L):
        h_prev = h
        xl_ref[...] = dot(h, llw_ref[l]) + llb_ref[l:l + 1, :]
        xr_ref[...] = dot(h, lrw_ref[l]) + lrb_ref[l:l + 1, :]
        etab = dot(ee_ref[:], lew_ref[l])
        catt = hexpand * att_ref[l:l + 1, :]
        catt_bf = catt.astype(jnp.bfloat16)

        def b_body(b, _, etab=etab, catt_bf=catt_bf):
            xl_b = xl_ref[pl.ds(b * _N, _N), :]
            irow = lax.broadcasted_iota(jnp.int32, (_N, 1), 0)
            lo_i = irow < (_N // 2)
            xlh = jnp.concatenate(
                [xl_b * hexpand[hh:hh + 1, :] for hh in range(_HEADS)],
                axis=0)

            for t in range(_N // _JT):
                j0 = t * _JT
                bf = jnp.bfloat16
                xr_t = xr_ref[pl.ds(b * _N + j0, _JT), :]
                xr_bf = xr_t.astype(bf)
                xlE = jnp.where(lo_i, xl_b + etab[t:t + 1, :],
                                xl_b + etab[2 + t:3 + t, :])
                xlE_bf = xlE.astype(bf)
                s = xr_bf[:, None, :] + xlE_bf[None, :, :]
                m = jnp.maximum(s, bf(0.2) * s)
                lt = dott(catt_bf, m.reshape(_JT * _N, _HID))
                lt3 = lt.reshape(_HEADS, _JT, _N)
                xl_t = xl_ref[pl.ds(b * _N + j0, _JT), :]
                sdr = xr_bf + (xl_t + etab[4 + t:5 + t, :]).astype(bf)
                mdr = jnp.maximum(sdr, bf(0.2) * sdr)
                sdw = xr_bf + (xl_t + etab[3 * t:3 * t + 1, :]).astype(bf)
                mdw = jnp.maximum(sdw, bf(0.2) * sdw)
                ldr = dott(catt_bf, mdr)
                ldw = dott(catt_bf, mdw)
                ex = jnp.exp(lt3)
                edr = jnp.exp(ldr)
                edw = jnp.exp(ldw)
                den = jnp.sum(ex, axis=2) - edw + edr
                rec = 1.0 / (den + 1e-16)
                alpha = ex * rec[:, :, None]
                alpha_cat = jnp.concatenate(
                    [alpha[hh] for hh in range(_HEADS)], axis=1)
                agg_t = dot(alpha_cat, xlh)
                dal = (edr - edw) * rec
                agg_t = agg_t + dot(dal.T, hexpand) * xl_t
                agg_ref[pl.ds(b * _N + j0, _JT), :] = agg_t
            return 0

        lax.fori_loop(0, _B, b_body, 0, unroll=_B)
        hh = agg_ref[:] + cb_ref[l:l + 1, :]
        hh = dot(hh, pw_ref[l]) + pb_ref[l:l + 1, :]
        hh = _ln(hh, lg_ref[l:l + 1, :], lb_ref[l:l + 1, :])
        hh = jnp.maximum(hh, 0.0)
        h = hh + h_prev

    hag = jnp.sum(h.reshape(_B, _N, _HID), axis=1)
    out_ref[:] = dot(hag, ow_ref[:]) + ob_ref[:]


def kernel(x, edge_emb, mlp_w1, mlp_b1, ln1_g, ln1_b, mlp_w2, mlp_b2, ln2_g,
           ln2_b, lin_l_w, lin_l_b, lin_r_w, lin_r_b, lin_e_w, att, conv_bias,
           proj_w, proj_b, lnl_g, lnl_b, out_w, out_b):
    f32 = jnp.float32
    x2 = x.reshape(_BN, 2).astype(f32)
    att_flat = att.reshape(_L, _HID)

    def row(v):
        return v.reshape(1, -1)

    return pl.pallas_call(
        _fwd,
        out_shape=jax.ShapeDtypeStruct((_B, 1), f32),
        scratch_shapes=[pltpu.VMEM((_BN, _HID), f32),
                        pltpu.VMEM((_BN, _HID), f32),
                        pltpu.VMEM((_BN, _HID), f32)],
    )(x2, edge_emb, mlp_w1, row(mlp_b1), row(ln1_g), row(ln1_b), mlp_w2,
      row(mlp_b2), row(ln2_g), row(ln2_b), lin_l_w, lin_l_b, lin_r_w,
      lin_r_b, lin_e_w, att_flat, conv_bias, proj_w, proj_b, lnl_g, lnl_b,
      out_w, out_b.reshape(1, 1))
